# baseline (speedup 1.0000x reference)
import functools
import math

import jax
import jax.numpy as jnp
from jax import lax
from jax.experimental import pallas as pl
from jax.experimental.pallas import tpu as pltpu
from jax.experimental.pallas import tpu_sc as plsc

N = 4096
V = 32000
_S = 0.1 / (V - 1)
_CONF = 0.9
_C_ROW = 0.1 * math.log(_S) + _CONF * math.log(_CONF)

_BR = 1024
_BC = 6400
_NRB = N // _BR
_NCB = V // _BC


def _tc_body(x_ref, a_ref, b_ref, m_ref, t_ref, out_ref, acc_ref):
    i = pl.program_id(0)
    j = pl.program_id(1)

    @pl.when((i == 0) & (j == 0))
    def _init():
        acc_ref[0] = 0.0
        acc_ref[1] = 0.0

    col = lax.broadcasted_iota(jnp.int32, (_BR, _BC), 1) + j * _BC
    w = jnp.where(col == t_ref[...], b_ref[...], a_ref[...])
    acc_ref[0] += jnp.sum(x_ref[...] * w)

    @pl.when(j == 0)
    def _count():
        acc_ref[1] += jnp.sum(m_ref[...])

    @pl.when((i == _NRB - 1) & (j == _NCB - 1))
    def _final():
        row = lax.broadcasted_iota(jnp.int32, (8, 128), 0)
        out_ref[...] = jnp.where(row == 1, acc_ref[1], acc_ref[0])


def _tc_weighted_sum(x, a, b, maskf, tgt2d, interpret=False):
    rowspec = pl.BlockSpec((_BR, 1), lambda i, j: (i, 0))
    return pl.pallas_call(
        _tc_body,
        grid=(_NRB, _NCB),
        in_specs=[
            pl.BlockSpec((_BR, _BC), lambda i, j: (i, j)),
            rowspec, rowspec, rowspec, rowspec,
        ],
        out_specs=pl.BlockSpec((8, 128), lambda i, j: (0, 0)),
        out_shape=jax.ShapeDtypeStruct((8, 128), jnp.float32),
        scratch_shapes=[pltpu.SMEM((2,), jnp.float32)],
        interpret=interpret,
    )(x, a, b, maskf, tgt2d)


_L = 16


@functools.cache
def _sc_finish_fn():
    mesh = plsc.VectorSubcoreMesh(core_axis_name="c", subcore_axis_name="s")

    @functools.partial(
        pl.kernel,
        mesh=mesh,
        out_type=jax.ShapeDtypeStruct((_L,), jnp.float32),
        scratch_types=[
            pltpu.VMEM((8, 128), jnp.float32),
            pltpu.VMEM((_L,), jnp.float32),
        ],
    )
    def _sc_finish(sacc_hbm, out_hbm, sacc_v, out_v):
        wid = lax.axis_index("s") * 2 + lax.axis_index("c")

        @pl.when(wid == 0)
        def _():
            pltpu.sync_copy(sacc_hbm, sacc_v)
            s_tot = sacc_v[0, pl.ds(0, _L)]
            m_cnt = sacc_v[1, pl.ds(0, _L)]
            out_v[...] = m_cnt * _C_ROW - s_tot
            pltpu.sync_copy(out_v, out_hbm)

    return _sc_finish


def kernel(x, target, target_mask):
    maskf = target_mask.astype(jnp.float32).reshape(N, 1)
    a = maskf * jnp.float32(_S)
    b = maskf * jnp.float32(_CONF)
    tgt2d = target.astype(jnp.int32).reshape(N, 1)
    sacc = _tc_weighted_sum(x, a, b, maskf, tgt2d)
    out = _sc_finish_fn()(sacc)
    return out[0]
